# baseline (speedup 1.0000x reference)
import jax
import jax.numpy as jnp
from jax import lax
from jax.experimental import pallas as pl
from jax.experimental.pallas import tpu as pltpu
from jax.experimental.pallas import tpu_sc as plsc

_N = 10000
_D = 128
_G = 64
_EPS = 1e-5

_NC = 2
_NS = 16
_NW = _NC * _NS
_CHUNK = 128
_NPAD = 10240
_ROWS_PER_TILE = _NPAD // _NS
_JUNK = _N
_RBLK = 1000
_F0 = 0.58


def _make_sc_aggr(ch0, ch1):
    chmax = max(ch0, ch1)

    def body(src_hbm, dst_hbm, h_hbm, z_hbm, p0_hbm, p1_hbm,
             aggr_sh, src_v, dst_v, rows_v, sem):
        c = lax.axis_index("c")
        s = lax.axis_index("s")
        wid = s * _NC + c
        base = s * _ROWS_PER_TILE

        pltpu.sync_copy(z_hbm, rows_v)
        for k in range(_ROWS_PER_TILE // _CHUNK):
            pltpu.sync_copy(rows_v, aggr_sh.at[pl.ds(base + k * _CHUNK, _CHUNK)])
        pltpu.sync_copy(src_hbm.at[wid], src_v)
        pltpu.sync_copy(dst_hbm.at[wid], dst_v)
        plsc.subcore_barrier()

        nch = jnp.where(c == 0, ch0, ch1)

        def chunk_body(j, carry):
            pltpu.async_copy(h_hbm.at[src_v.at[j]], rows_v, sem).wait()
            pltpu.sync_copy(rows_v, aggr_sh.at[dst_v.at[j]], add=True)
            return carry

        lax.fori_loop(0, nch, chunk_body, 0)

        plsc.subcore_barrier()
        for k in range(_ROWS_PER_TILE // _CHUNK):
            sl = pl.ds(base + k * _CHUNK, _CHUNK)
            pltpu.sync_copy(aggr_sh.at[sl], rows_v)

            @pl.when(c == 0)
            def _():
                pltpu.sync_copy(rows_v, p0_hbm.at[sl])

            @pl.when(c == 1)
            def _():
                pltpu.sync_copy(rows_v, p1_hbm.at[sl])

    mesh = plsc.VectorSubcoreMesh(
        core_axis_name="c", subcore_axis_name="s",
        num_cores=_NC, num_subcores=_NS)
    return pl.kernel(
        body,
        out_type=[jax.ShapeDtypeStruct((_NPAD, _D), jnp.float32),
                  jax.ShapeDtypeStruct((_NPAD, _D), jnp.float32)],
        mesh=mesh,
        scratch_types=[
            pltpu.VMEM_SHARED((_NPAD, _D), jnp.float32),
            pltpu.VMEM((chmax, _CHUNK), jnp.int32),
            pltpu.VMEM((chmax, _CHUNK), jnp.int32),
            pltpu.VMEM((_CHUNK, _D), jnp.float32),
            pltpu.SemaphoreType.DMA,
        ],
    )


def _mlp_body(h_ref, p0_ref, p1_ref, w1_ref, b1_ref, w2_ref, b2_ref,
              mid_ref, sum_ref, sq_ref):
    u = h_ref[...] + p0_ref[...] + p1_ref[...]
    t = jnp.dot(u, w1_ref[...], preferred_element_type=jnp.float32) + b1_ref[...]
    t = jnp.maximum(t, 0.0)
    mid = jnp.dot(t, w2_ref[...], preferred_element_type=jnp.float32) + b2_ref[...]
    mid_ref[...] = mid

    @pl.when(pl.program_id(0) == 0)
    def _():
        sum_ref[...] = jnp.zeros_like(sum_ref)
        sq_ref[...] = jnp.zeros_like(sq_ref)

    sum_ref[...] += jnp.sum(mid, axis=0, keepdims=True)
    sq_ref[...] += jnp.sum(mid * mid, axis=0, keepdims=True)


def _bn_scale(sum_v, sq_v, g_v, be_v):
    mean = sum_v * (1.0 / _N)
    var = sq_v * (1.0 / _N) - mean * mean
    a = g_v * lax.rsqrt(var + _EPS)
    b = be_v - mean * a
    return a, b


def _bn_body(mid_ref, sum_ref, sq_ref, g_ref, be_ref, out_ref):
    a, b = _bn_scale(sum_ref[...], sq_ref[...], g_ref[...], be_ref[...])
    out_ref[...] = jnp.maximum(mid_ref[...] * a + b, 0.0)


def _bn_pool_body(mid_ref, sum_ref, sq_ref, g_ref, be_ref, batch_ref,
                  node_ref, graph_ref, cnt_ref):
    a, b = _bn_scale(sum_ref[...], sq_ref[...], g_ref[...], be_ref[...])
    hblk = jnp.maximum(mid_ref[...] * a + b, 0.0)
    node_ref[...] = hblk
    bb = batch_ref[...]
    m = (bb == lax.broadcasted_iota(jnp.int32, (_RBLK, _G), 1)
         ).astype(jnp.float32)
    dn = (((0,), (0,)), ((), ()))
    sums = lax.dot_general(m, hblk, dn, preferred_element_type=jnp.float32)
    cnts = lax.dot_general(m, jnp.ones_like(hblk), dn,
                           preferred_element_type=jnp.float32)

    @pl.when(pl.program_id(0) == 0)
    def _():
        graph_ref[...] = jnp.zeros_like(graph_ref)
        cnt_ref[...] = jnp.zeros_like(cnt_ref)

    graph_ref[...] += sums
    cnt_ref[...] += cnts

    @pl.when(pl.program_id(0) == pl.num_programs(0) - 1)
    def _():
        graph_ref[...] = graph_ref[...] / jnp.maximum(cnt_ref[...], 1.0)


def _row_spec(blk):
    return pl.BlockSpec(blk, lambda i: (i, 0))


def _const_spec(blk):
    return pl.BlockSpec(blk, lambda i: (0, 0))


def _mlp_call(h, p0, p1, w1, b1, w2, b2):
    grid = (_N // _RBLK,)
    return pl.pallas_call(
        _mlp_body,
        grid=grid,
        in_specs=[
            _row_spec((_RBLK, _D)), _row_spec((_RBLK, _D)),
            _row_spec((_RBLK, _D)),
            _const_spec((_D, _D)), _const_spec((1, _D)),
            _const_spec((_D, _D)), _const_spec((1, _D)),
        ],
        out_specs=[_row_spec((_RBLK, _D)), _const_spec((1, _D)),
                   _const_spec((1, _D))],
        out_shape=[jax.ShapeDtypeStruct((_N, _D), jnp.float32),
                   jax.ShapeDtypeStruct((1, _D), jnp.float32),
                   jax.ShapeDtypeStruct((1, _D), jnp.float32)],
    )(h, p0, p1, w1, b1, w2, b2)


def _bn_call(mid, s1, s2, g, be):
    return pl.pallas_call(
        _bn_body,
        grid=(_N // _RBLK,),
        in_specs=[_row_spec((_RBLK, _D)), _const_spec((1, _D)),
                  _const_spec((1, _D)), _const_spec((1, _D)),
                  _const_spec((1, _D))],
        out_specs=_row_spec((_RBLK, _D)),
        out_shape=jax.ShapeDtypeStruct((_N, _D), jnp.float32),
    )(mid, s1, s2, g, be)


def _bn_pool_call(mid, s1, s2, g, be, batch2):
    return pl.pallas_call(
        _bn_pool_body,
        grid=(_N // _RBLK,),
        in_specs=[_row_spec((_RBLK, _D)), _const_spec((1, _D)),
                  _const_spec((1, _D)), _const_spec((1, _D)),
                  _const_spec((1, _D)), _row_spec((_RBLK, 1))],
        out_specs=[_row_spec((_RBLK, _D)), _const_spec((_G, _D))],
        out_shape=[jax.ShapeDtypeStruct((_N, _D), jnp.float32),
                   jax.ShapeDtypeStruct((_G, _D), jnp.float32)],
        scratch_shapes=[pltpu.VMEM((_G, _D), jnp.float32)],
    )(mid, s1, s2, g, be, batch2)


def kernel(x, edge_index, batch, params):
    e = edge_index.shape[1]
    t = -(-e // _CHUNK)
    ch0 = max(1, int(round(t * _F0 / _NS)))
    ch1 = max(1, -(-max(t - _NS * ch0, 0) // _NS))
    chmax = max(ch0, ch1)
    cap0 = _NS * ch0 * _CHUNK
    total = _NS * (ch0 + ch1) * _CHUNK
    src = jnp.concatenate(
        [edge_index[0], jnp.zeros((total - e,), jnp.int32)])
    dst = jnp.concatenate(
        [edge_index[1], jnp.full((total - e,), _JUNK, jnp.int32)])
    a0s = jnp.pad(src[:cap0].reshape(_NS, ch0, _CHUNK),
                  ((0, 0), (0, chmax - ch0), (0, 0)))
    a1s = jnp.pad(src[cap0:].reshape(_NS, ch1, _CHUNK),
                  ((0, 0), (0, chmax - ch1), (0, 0)))
    a0d = jnp.pad(dst[:cap0].reshape(_NS, ch0, _CHUNK),
                  ((0, 0), (0, chmax - ch0), (0, 0)),
                  constant_values=_JUNK)
    a1d = jnp.pad(dst[cap0:].reshape(_NS, ch1, _CHUNK),
                  ((0, 0), (0, chmax - ch1), (0, 0)),
                  constant_values=_JUNK)
    src3 = jnp.stack([a0s, a1s], axis=1).reshape(_NW, chmax, _CHUNK)
    dst3 = jnp.stack([a0d, a1d], axis=1).reshape(_NW, chmax, _CHUNK)
    zeros = jnp.zeros((_CHUNK, _D), jnp.float32)
    batch2 = batch.reshape(_N, 1)

    sc_aggr = _make_sc_aggr(ch0, ch1)

    h = x
    node = graph = None
    n_layers = len(params)
    for i, (w1, b1, w2, b2, gamma, beta) in enumerate(params):
        p0, p1 = sc_aggr(src3, dst3, h, zeros)
        mid, s1, s2 = _mlp_call(h, p0, p1, w1, b1.reshape(1, _D),
                                w2, b2.reshape(1, _D))
        g2 = gamma.reshape(1, _D)
        be2 = beta.reshape(1, _D)
        if i < n_layers - 1:
            h = _bn_call(mid, s1, s2, g2, be2)
        else:
            node, graph = _bn_pool_call(mid, s1, s2, g2, be2, batch2)
    return (node, graph)


# speedup vs baseline: 1.0298x; 1.0298x over previous
import jax
import jax.numpy as jnp
from jax import lax
from jax.experimental import pallas as pl
from jax.experimental.pallas import tpu as pltpu
from jax.experimental.pallas import tpu_sc as plsc

_N = 10000
_D = 128
_G = 64
_EPS = 1e-5

_NC = 2
_NS = 16
_NW = _NC * _NS
_CHUNK = 128
_NPAD = 10240
_ROWS_PER_TILE = _NPAD // _NS
_JUNK = _N
_RBLK = 1000
_F0 = 0.62


def _make_sc_aggr(ch0, ch1):
    chmax = max(ch0, ch1)

    def body(src_hbm, dst_hbm, h_hbm, z_hbm, p0_hbm, p1_hbm,
             aggr_sh, src_v, dst_v, rows_v, sem):
        c = lax.axis_index("c")
        s = lax.axis_index("s")
        wid = s * _NC + c
        base = s * _ROWS_PER_TILE

        pltpu.sync_copy(z_hbm, rows_v)
        for k in range(_ROWS_PER_TILE // _CHUNK):
            pltpu.sync_copy(rows_v, aggr_sh.at[pl.ds(base + k * _CHUNK, _CHUNK)])
        pltpu.sync_copy(src_hbm.at[wid], src_v)
        pltpu.sync_copy(dst_hbm.at[wid], dst_v)
        plsc.subcore_barrier()

        nch = jnp.where(c == 0, ch0, ch1)

        def chunk_body(j, carry):
            pltpu.async_copy(h_hbm.at[src_v.at[j]], rows_v, sem).wait()
            pltpu.sync_copy(rows_v, aggr_sh.at[dst_v.at[j]], add=True)
            return carry

        lax.fori_loop(0, nch, chunk_body, 0)

        plsc.subcore_barrier()
        for k in range(_ROWS_PER_TILE // _CHUNK):
            sl = pl.ds(base + k * _CHUNK, _CHUNK)
            pltpu.sync_copy(aggr_sh.at[sl], rows_v)

            @pl.when(c == 0)
            def _():
                pltpu.sync_copy(rows_v, p0_hbm.at[sl])

            @pl.when(c == 1)
            def _():
                pltpu.sync_copy(rows_v, p1_hbm.at[sl])

    mesh = plsc.VectorSubcoreMesh(
        core_axis_name="c", subcore_axis_name="s",
        num_cores=_NC, num_subcores=_NS)
    return pl.kernel(
        body,
        out_type=[jax.ShapeDtypeStruct((_NPAD, _D), jnp.float32),
                  jax.ShapeDtypeStruct((_NPAD, _D), jnp.float32)],
        mesh=mesh,
        scratch_types=[
            pltpu.VMEM_SHARED((_NPAD, _D), jnp.float32),
            pltpu.VMEM((chmax, _CHUNK), jnp.int32),
            pltpu.VMEM((chmax, _CHUNK), jnp.int32),
            pltpu.VMEM((_CHUNK, _D), jnp.float32),
            pltpu.SemaphoreType.DMA,
        ],
    )


def _mlp_body(h_ref, p0_ref, p1_ref, w1_ref, b1_ref, w2_ref, b2_ref,
              mid_ref, sum_ref, sq_ref):
    u = h_ref[...] + p0_ref[...] + p1_ref[...]
    t = jnp.dot(u, w1_ref[...], preferred_element_type=jnp.float32) + b1_ref[...]
    t = jnp.maximum(t, 0.0)
    mid = jnp.dot(t, w2_ref[...], preferred_element_type=jnp.float32) + b2_ref[...]
    mid_ref[...] = mid

    @pl.when(pl.program_id(0) == 0)
    def _():
        sum_ref[...] = jnp.zeros_like(sum_ref)
        sq_ref[...] = jnp.zeros_like(sq_ref)

    sum_ref[...] += jnp.sum(mid, axis=0, keepdims=True)
    sq_ref[...] += jnp.sum(mid * mid, axis=0, keepdims=True)


def _bn_scale(sum_v, sq_v, g_v, be_v):
    mean = sum_v * (1.0 / _N)
    var = sq_v * (1.0 / _N) - mean * mean
    a = g_v * lax.rsqrt(var + _EPS)
    b = be_v - mean * a
    return a, b


def _bn_body(mid_ref, sum_ref, sq_ref, g_ref, be_ref, out_ref):
    a, b = _bn_scale(sum_ref[...], sq_ref[...], g_ref[...], be_ref[...])
    out_ref[...] = jnp.maximum(mid_ref[...] * a + b, 0.0)


def _bn_pool_body(mid_ref, sum_ref, sq_ref, g_ref, be_ref, batch_ref,
                  node_ref, graph_ref, cnt_ref):
    a, b = _bn_scale(sum_ref[...], sq_ref[...], g_ref[...], be_ref[...])
    hblk = jnp.maximum(mid_ref[...] * a + b, 0.0)
    node_ref[...] = hblk
    bb = batch_ref[...]
    m = (bb == lax.broadcasted_iota(jnp.int32, (_RBLK, _G), 1)
         ).astype(jnp.float32)
    dn = (((0,), (0,)), ((), ()))
    sums = lax.dot_general(m, hblk, dn, preferred_element_type=jnp.float32)
    cnts = lax.dot_general(m, jnp.ones_like(hblk), dn,
                           preferred_element_type=jnp.float32)

    @pl.when(pl.program_id(0) == 0)
    def _():
        graph_ref[...] = jnp.zeros_like(graph_ref)
        cnt_ref[...] = jnp.zeros_like(cnt_ref)

    graph_ref[...] += sums
    cnt_ref[...] += cnts

    @pl.when(pl.program_id(0) == pl.num_programs(0) - 1)
    def _():
        graph_ref[...] = graph_ref[...] / jnp.maximum(cnt_ref[...], 1.0)


def _row_spec(blk):
    return pl.BlockSpec(blk, lambda i: (i, 0))


def _const_spec(blk):
    return pl.BlockSpec(blk, lambda i: (0, 0))


def _mlp_call(h, p0, p1, w1, b1, w2, b2):
    grid = (_N // _RBLK,)
    return pl.pallas_call(
        _mlp_body,
        grid=grid,
        in_specs=[
            _row_spec((_RBLK, _D)), _row_spec((_RBLK, _D)),
            _row_spec((_RBLK, _D)),
            _const_spec((_D, _D)), _const_spec((1, _D)),
            _const_spec((_D, _D)), _const_spec((1, _D)),
        ],
        out_specs=[_row_spec((_RBLK, _D)), _const_spec((1, _D)),
                   _const_spec((1, _D))],
        out_shape=[jax.ShapeDtypeStruct((_N, _D), jnp.float32),
                   jax.ShapeDtypeStruct((1, _D), jnp.float32),
                   jax.ShapeDtypeStruct((1, _D), jnp.float32)],
    )(h, p0, p1, w1, b1, w2, b2)


def _bn_call(mid, s1, s2, g, be):
    return pl.pallas_call(
        _bn_body,
        grid=(_N // _RBLK,),
        in_specs=[_row_spec((_RBLK, _D)), _const_spec((1, _D)),
                  _const_spec((1, _D)), _const_spec((1, _D)),
                  _const_spec((1, _D))],
        out_specs=_row_spec((_RBLK, _D)),
        out_shape=jax.ShapeDtypeStruct((_N, _D), jnp.float32),
    )(mid, s1, s2, g, be)


def _bn_pool_call(mid, s1, s2, g, be, batch2):
    return pl.pallas_call(
        _bn_pool_body,
        grid=(_N // _RBLK,),
        in_specs=[_row_spec((_RBLK, _D)), _const_spec((1, _D)),
                  _const_spec((1, _D)), _const_spec((1, _D)),
                  _const_spec((1, _D)), _row_spec((_RBLK, 1))],
        out_specs=[_row_spec((_RBLK, _D)), _const_spec((_G, _D))],
        out_shape=[jax.ShapeDtypeStruct((_N, _D), jnp.float32),
                   jax.ShapeDtypeStruct((_G, _D), jnp.float32)],
        scratch_shapes=[pltpu.VMEM((_G, _D), jnp.float32)],
    )(mid, s1, s2, g, be, batch2)


def kernel(x, edge_index, batch, params):
    e = edge_index.shape[1]
    t = -(-e // _CHUNK)
    ch0 = max(1, int(round(t * _F0 / _NS)))
    ch1 = max(1, -(-max(t - _NS * ch0, 0) // _NS))
    chmax = max(ch0, ch1)
    cap0 = _NS * ch0 * _CHUNK
    total = _NS * (ch0 + ch1) * _CHUNK
    src = jnp.concatenate(
        [edge_index[0], jnp.zeros((total - e,), jnp.int32)])
    dst = jnp.concatenate(
        [edge_index[1], jnp.full((total - e,), _JUNK, jnp.int32)])
    a0s = jnp.pad(src[:cap0].reshape(_NS, ch0, _CHUNK),
                  ((0, 0), (0, chmax - ch0), (0, 0)))
    a1s = jnp.pad(src[cap0:].reshape(_NS, ch1, _CHUNK),
                  ((0, 0), (0, chmax - ch1), (0, 0)))
    a0d = jnp.pad(dst[:cap0].reshape(_NS, ch0, _CHUNK),
                  ((0, 0), (0, chmax - ch0), (0, 0)),
                  constant_values=_JUNK)
    a1d = jnp.pad(dst[cap0:].reshape(_NS, ch1, _CHUNK),
                  ((0, 0), (0, chmax - ch1), (0, 0)),
                  constant_values=_JUNK)
    src3 = jnp.stack([a0s, a1s], axis=1).reshape(_NW, chmax, _CHUNK)
    dst3 = jnp.stack([a0d, a1d], axis=1).reshape(_NW, chmax, _CHUNK)
    zeros = jnp.zeros((_CHUNK, _D), jnp.float32)
    batch2 = batch.reshape(_N, 1)

    sc_aggr = _make_sc_aggr(ch0, ch1)

    h = x
    node = graph = None
    n_layers = len(params)
    for i, (w1, b1, w2, b2, gamma, beta) in enumerate(params):
        p0, p1 = sc_aggr(src3, dst3, h, zeros)
        mid, s1, s2 = _mlp_call(h, p0, p1, w1, b1.reshape(1, _D),
                                w2, b2.reshape(1, _D))
        g2 = gamma.reshape(1, _D)
        be2 = beta.reshape(1, _D)
        if i < n_layers - 1:
            h = _bn_call(mid, s1, s2, g2, be2)
        else:
            node, graph = _bn_pool_call(mid, s1, s2, g2, be2, batch2)
    return (node, graph)
